# Initial kernel scaffold; baseline (speedup 1.0000x reference)
import jax
import jax.numpy as jnp
from jax.experimental import pallas as pl


def kernel(features, rows, cols, W1, b1, W2, b2, Wo, bo, Wr, br):
    raise NotImplementedError("write your pallas kernel here")


# speedup vs baseline: 5.3605x; 5.3605x over previous
import functools

import jax
import jax.numpy as jnp
from jax import lax
from jax.experimental import pallas as pl
from jax.experimental.pallas import tpu as pltpu
from jax.experimental.pallas import tpu_sc as plsc

N, D, E, MID, OUT = 10000, 128, 320000, 128, 128

NC, NS, L = 2, 16, 16
NW = NC * NS
EPW = E // NW
K = 80
NCHUNK = EPW // K
NBLK = N // K
BPT = -(-NBLK // NS)

ROW_BLK = 1000
GRID = N // ROW_BLK


def _mm3_body(x_ref, w1_ref, b1_ref, w2_ref, b2_ref, wr_ref, br_ref,
              m1_ref, m2_ref, h1_ref):
    x = x_ref[...]
    m1_ref[...] = jnp.dot(x, w1_ref[...], preferred_element_type=jnp.float32) + b1_ref[...]
    m2_ref[...] = jnp.dot(x, w2_ref[...], preferred_element_type=jnp.float32) + b2_ref[...]
    h1_ref[...] = jnp.dot(x, wr_ref[...], preferred_element_type=jnp.float32) + br_ref[...]


def _mm3(x, w1, b1, w2, b2, wr, br):
    blk = pl.BlockSpec((ROW_BLK, D), lambda i: (i, 0))
    wspec = pl.BlockSpec((D, MID), lambda i: (0, 0))
    bspec = pl.BlockSpec((1, MID), lambda i: (0, 0))
    return pl.pallas_call(
        _mm3_body,
        grid=(GRID,),
        in_specs=[blk, wspec, bspec, wspec, bspec, wspec, bspec],
        out_specs=[blk, blk, blk],
        out_shape=[jax.ShapeDtypeStruct((N, MID), jnp.float32)] * 3,
    )(x, w1, b1, w2, b2, wr, br)


def _final_body(h1_ref, m_ref, wo_ref, bo_ref, out_ref):
    msgs = m_ref[0] + m_ref[1]
    h2 = jnp.dot(msgs, wo_ref[...], preferred_element_type=jnp.float32) + bo_ref[...]
    out_ref[...] = jnp.maximum(h1_ref[...] + h2, 0.0)


def _final(h1, partials, wo, bo):
    blk = pl.BlockSpec((ROW_BLK, D), lambda i: (i, 0))
    return pl.pallas_call(
        _final_body,
        grid=(GRID,),
        in_specs=[
            blk,
            pl.BlockSpec((2, ROW_BLK, MID), lambda i: (0, i, 0)),
            pl.BlockSpec((MID, OUT), lambda i: (0, 0)),
            pl.BlockSpec((1, OUT), lambda i: (0, 0)),
        ],
        out_specs=blk,
        out_shape=jax.ShapeDtypeStruct((N, OUT), jnp.float32),
    )(h1, partials, wo, bo)


def _edge_body(msg1_hbm, msg2_hbm, rows_hbm, cols_hbm, out_hbm,
               ridx, cidx, g1, g2, acc, sem1, sem2):
    c = lax.axis_index("c")
    s = lax.axis_index("s")
    wid = s * NC + c

    zeros = jnp.zeros((L,), jnp.float32)

    def _zero_row(r, _):
        for j in range(D // L):
            g1[r, pl.ds(j * L, L)] = zeros
        return 0

    lax.fori_loop(0, K, _zero_row, 0)

    for jb in range(BPT):
        b = s + jb * NS

        @pl.when(b < NBLK)
        def _():
            pltpu.sync_copy(g1, acc.at[pl.ds(b * K, K)])

    plsc.subcore_barrier()

    def _chunk(i, _):
        base = wid * EPW + i * K
        pltpu.sync_copy(rows_hbm.at[pl.ds(base, K)], ridx.at[0])
        pltpu.sync_copy(cols_hbm.at[pl.ds(base, K)], cidx)
        cp1 = pltpu.async_copy(msg1_hbm.at[ridx.at[0]], g1, sem1)
        cp2 = pltpu.async_copy(msg2_hbm.at[cidx], g2, sem2)
        cp1.wait()
        cp2.wait()

        def _row(r, _):
            for j in range(D // L):
                sl = pl.ds(j * L, L)
                g1[r, sl] = jnp.maximum(g1[r, sl] + g2[r, sl], 0.0)
            return 0

        lax.fori_loop(0, K, _row, 0)
        pltpu.sync_copy(g1, acc.at[ridx.at[0]], add=True)
        return 0

    lax.fori_loop(0, NCHUNK, _chunk, 0)
    plsc.subcore_barrier()

    for jb in range(BPT):
        b = s + jb * NS

        @pl.when(b < NBLK)
        def _():
            pltpu.sync_copy(acc.at[pl.ds(b * K, K)],
                            out_hbm.at[c, pl.ds(b * K, K)])


@functools.partial(
    pl.kernel,
    out_type=jax.ShapeDtypeStruct((NC, N, MID), jnp.float32),
    mesh=plsc.VectorSubcoreMesh(core_axis_name="c", subcore_axis_name="s"),
    scratch_types=[
        pltpu.VMEM((1, K), jnp.int32),
        pltpu.VMEM((K,), jnp.int32),
        pltpu.VMEM((K, MID), jnp.float32),
        pltpu.VMEM((K, MID), jnp.float32),
        pltpu.VMEM_SHARED((N, MID), jnp.float32),
        pltpu.SemaphoreType.DMA,
        pltpu.SemaphoreType.DMA,
    ],
)
def _edge_sc(msg1_hbm, msg2_hbm, rows_hbm, cols_hbm, out_hbm,
             ridx, cidx, g1, g2, acc, sem1, sem2):
    _edge_body(msg1_hbm, msg2_hbm, rows_hbm, cols_hbm, out_hbm,
               ridx, cidx, g1, g2, acc, sem1, sem2)


def kernel(features, rows, cols, W1, b1, W2, b2, Wo, bo, Wr, br):
    msg1, msg2, h1 = _mm3(features, W1, b1.reshape(1, MID),
                          W2, b2.reshape(1, MID), Wr, br.reshape(1, OUT))
    partials = _edge_sc(msg1, msg2, rows, cols)
    return _final(h1, partials, Wo, bo.reshape(1, OUT))


# speedup vs baseline: 8.4656x; 1.5793x over previous
import functools

import jax
import jax.numpy as jnp
from jax import lax
from jax.experimental import pallas as pl
from jax.experimental.pallas import tpu as pltpu
from jax.experimental.pallas import tpu_sc as plsc

N, D, E, MID, OUT = 10000, 128, 320000, 128, 128

NC, NS, L = 2, 16, 16
H = MID // NC
EPW = E // NS
K = 80
NCHUNK = EPW // K
NBUF = 5
NBLK = N // K
BPT = -(-NBLK // NS)

ROW_BLK = 1000
GRID = N // ROW_BLK


def _mm3_body(x_ref, w1_ref, b1_ref, w2_ref, b2_ref, wr_ref, br_ref,
              m1_ref, m2_ref, h1_ref):
    x = x_ref[...]
    m1 = jnp.dot(x, w1_ref[...], preferred_element_type=jnp.float32) + b1_ref[...]
    m2 = jnp.dot(x, w2_ref[...], preferred_element_type=jnp.float32) + b2_ref[...]
    m1_ref[0] = m1[:, :H]
    m1_ref[1] = m1[:, H:]
    m2_ref[0] = m2[:, :H]
    m2_ref[1] = m2[:, H:]
    h1_ref[...] = jnp.dot(x, wr_ref[...], preferred_element_type=jnp.float32) + br_ref[...]


def _mm3(x, w1, b1, w2, b2, wr, br):
    blk = pl.BlockSpec((ROW_BLK, D), lambda i: (i, 0))
    hblk = pl.BlockSpec((NC, ROW_BLK, H), lambda i: (0, i, 0))
    wspec = pl.BlockSpec((D, MID), lambda i: (0, 0))
    bspec = pl.BlockSpec((1, MID), lambda i: (0, 0))
    return pl.pallas_call(
        _mm3_body,
        grid=(GRID,),
        in_specs=[blk, wspec, bspec, wspec, bspec, wspec, bspec],
        out_specs=[hblk, hblk, blk],
        out_shape=[
            jax.ShapeDtypeStruct((NC, N, H), jnp.float32),
            jax.ShapeDtypeStruct((NC, N, H), jnp.float32),
            jax.ShapeDtypeStruct((N, MID), jnp.float32),
        ],
    )(x, w1, b1, w2, b2, wr, br)


def _final_body(h1_ref, m_ref, wo_ref, bo_ref, out_ref):
    msgs = jnp.concatenate([m_ref[0], m_ref[1]], axis=-1)
    h2 = jnp.dot(msgs, wo_ref[...], preferred_element_type=jnp.float32) + bo_ref[...]
    out_ref[...] = jnp.maximum(h1_ref[...] + h2, 0.0)


def _final(h1, msgs_halves, wo, bo):
    blk = pl.BlockSpec((ROW_BLK, D), lambda i: (i, 0))
    return pl.pallas_call(
        _final_body,
        grid=(GRID,),
        in_specs=[
            blk,
            pl.BlockSpec((NC, ROW_BLK, H), lambda i: (0, i, 0)),
            pl.BlockSpec((MID, OUT), lambda i: (0, 0)),
            pl.BlockSpec((1, OUT), lambda i: (0, 0)),
        ],
        out_specs=blk,
        out_shape=jax.ShapeDtypeStruct((N, OUT), jnp.float32),
    )(h1, msgs_halves, wo, bo)


def _edge_body(m1_hbm, m2_hbm, rows_hbm, cols_hbm, out_hbm,
               ridx, cidx, g1, g2, acc, semi, semr, semc, sems):
    c = lax.axis_index("c")
    s = lax.axis_index("s")

    zeros = jnp.zeros((L,), jnp.float32)

    def _zero_row(r, _):
        for j in range(H // L):
            g1[0, r, pl.ds(j * L, L)] = zeros
        return 0

    lax.fori_loop(0, K, _zero_row, 0)

    for jb in range(BPT):
        b = s + jb * NS

        @pl.when(b < NBLK)
        def _():
            pltpu.sync_copy(g1.at[0], acc.at[pl.ds(b * K, K)])

    plsc.subcore_barrier()

    m1h = m1_hbm.at[c]
    m2h = m2_hbm.at[c]

    def _group(r, _):
        for b in range(NBUF):
            base = (r * NBUF + b) * K + s * EPW

            @pl.when(r > 0)
            def _():
                pltpu.make_async_copy(g1.at[b], acc.at[ridx.at[b]], sems.at[b]).wait()

            pltpu.async_copy(rows_hbm.at[pl.ds(base, K)], ridx.at[b], semi.at[b])
            pltpu.async_copy(cols_hbm.at[pl.ds(base, K)], cidx.at[b], semi.at[b])
        for b in range(NBUF):
            base = (r * NBUF + b) * K + s * EPW
            pltpu.make_async_copy(rows_hbm.at[pl.ds(base, K)], ridx.at[b], semi.at[b]).wait()
            pltpu.make_async_copy(cols_hbm.at[pl.ds(base, K)], cidx.at[b], semi.at[b]).wait()
            pltpu.async_copy(m1h.at[ridx.at[b]], g1.at[b], semr.at[b])
            pltpu.async_copy(m2h.at[cidx.at[b]], g2.at[b], semc.at[b])
        for b in range(NBUF):
            pltpu.make_async_copy(m1h.at[ridx.at[b]], g1.at[b], semr.at[b]).wait()
            pltpu.make_async_copy(m2h.at[cidx.at[b]], g2.at[b], semc.at[b]).wait()

            def _row(rr, _):
                for j in range(H // L):
                    sl = pl.ds(j * L, L)
                    g1[b, rr, sl] = jnp.maximum(g1[b, rr, sl] + g2[b, rr, sl], 0.0)
                return 0

            lax.fori_loop(0, K, _row, 0)
            pltpu.async_copy(g1.at[b], acc.at[ridx.at[b]], sems.at[b], add=True)
        return 0

    lax.fori_loop(0, NCHUNK // NBUF, _group, 0)
    for b in range(NBUF):
        pltpu.make_async_copy(g1.at[b], acc.at[ridx.at[b]], sems.at[b]).wait()
    plsc.subcore_barrier()

    for jb in range(BPT):
        b = s + jb * NS

        @pl.when(b < NBLK)
        def _():
            pltpu.sync_copy(acc.at[pl.ds(b * K, K)],
                            out_hbm.at[c, pl.ds(b * K, K)])


@functools.partial(
    pl.kernel,
    out_type=jax.ShapeDtypeStruct((NC, N, H), jnp.float32),
    mesh=plsc.VectorSubcoreMesh(core_axis_name="c", subcore_axis_name="s"),
    compiler_params=pltpu.CompilerParams(use_tc_tiling_on_sc=False),
    scratch_types=[
        pltpu.VMEM((NBUF, K), jnp.int32),
        pltpu.VMEM((NBUF, K), jnp.int32),
        pltpu.VMEM((NBUF, K, H), jnp.float32),
        pltpu.VMEM((NBUF, K, H), jnp.float32),
        pltpu.VMEM_SHARED((N, H), jnp.float32),
        pltpu.SemaphoreType.DMA((NBUF,)),
        pltpu.SemaphoreType.DMA((NBUF,)),
        pltpu.SemaphoreType.DMA((NBUF,)),
        pltpu.SemaphoreType.DMA((NBUF,)),
    ],
)
def _edge_sc(m1_hbm, m2_hbm, rows_hbm, cols_hbm, out_hbm,
             ridx, cidx, g1, g2, acc, semi, semr, semc, sems):
    _edge_body(m1_hbm, m2_hbm, rows_hbm, cols_hbm, out_hbm,
               ridx, cidx, g1, g2, acc, semi, semr, semc, sems)


def kernel(features, rows, cols, W1, b1, W2, b2, Wo, bo, Wr, br):
    m1s, m2s, h1 = _mm3(features, W1, b1.reshape(1, MID),
                        W2, b2.reshape(1, MID), Wr, br.reshape(1, OUT))
    msgs_halves = _edge_sc(m1s, m2s, rows, cols)
    return _final(h1, msgs_halves, Wo, bo.reshape(1, OUT))
